# baseline (speedup 1.0000x reference)
import functools

import jax
import jax.numpy as jnp
from jax import lax
from jax.experimental import pallas as pl
from jax.experimental.pallas import tpu as pltpu
from jax.experimental.pallas import tpu_sc as plsc

_NC = 2
_NS = 16
_NW = _NC * _NS
_CHUNK = 128


def _sc_gather_body(P, T, cpt_hbm, ht_hbm, table_hbm,
                    out_cpt_hbm, out_ht_hbm, idx_v, rows_v, sem, sem2):
    ppw = P // _NW
    npc = ppw // _CHUNK
    ncc = T * npc
    nhc = 2 * ppw // _CHUNK
    n_chunks = ncc + nhc
    wid = lax.axis_index("s") * _NC + lax.axis_index("c")
    p0 = wid * ppw

    def stage():
        for t in range(T):
            yield (cpt_hbm.at[pl.ds(t * P + p0, ppw)],
                   idx_v.at[pl.ds(t * ppw, ppw)])
        yield (ht_hbm.at[pl.ds(p0 * 2, ppw * 2)],
               idx_v.at[pl.ds(T * ppw, 2 * ppw)])

    for src, dst in stage():
        pltpu.async_copy(src, dst, sem2)
    for src, dst in stage():
        pltpu.make_async_copy(src, dst, sem2).wait()

    def gather(c):
        b = lax.rem(c, 2)
        pltpu.async_copy(table_hbm.at[idx_v.at[pl.ds(c * _CHUNK, _CHUNK)]],
                         rows_v.at[b], sem)

    gather(0)

    def body(c, carry):
        @pl.when(c + 1 < n_chunks)
        def _():
            gather(c + 1)

        b = lax.rem(c, 2)
        pltpu.make_async_copy(
            table_hbm.at[idx_v.at[pl.ds(c * _CHUNK, _CHUNK)]],
            rows_v.at[b], sem).wait()

        @pl.when(c < ncc)
        def _():
            t = c // npc
            pb = lax.rem(c, npc)
            pltpu.sync_copy(
                rows_v.at[b],
                out_cpt_hbm.at[pl.ds(t * P + p0 + pb * _CHUNK, _CHUNK)])

        @pl.when(c >= ncc)
        def _():
            pltpu.sync_copy(
                rows_v.at[b],
                out_ht_hbm.at[pl.ds(2 * p0 + (c - ncc) * _CHUNK, _CHUNK)])

        return carry

    lax.fori_loop(0, n_chunks, body, 0)


def _sc_gather(cpt_tm, ht_flat, table, P, T):
    D = table.shape[1]
    ppw = P // _NW
    assert ppw % _CHUNK == 0 and (T * ppw) % _CHUNK == 0
    mesh = plsc.VectorSubcoreMesh(core_axis_name="c", subcore_axis_name="s")
    k = functools.partial(
        pl.kernel,
        mesh=mesh,
        out_type=(
            jax.ShapeDtypeStruct((T * P, D), table.dtype),
            jax.ShapeDtypeStruct((2 * P, D), table.dtype),
        ),
        scratch_types=[
            pltpu.VMEM(((T + 2) * ppw,), jnp.int32),
            pltpu.VMEM((2, _CHUNK, D), table.dtype),
            pltpu.SemaphoreType.DMA,
            pltpu.SemaphoreType.DMA,
        ],
    )(functools.partial(_sc_gather_body, P, T))
    return k(cpt_tm, ht_flat, table)


def _enc_body(T, cpt_ref, meta_ref, ht_ref, rel128_ref, wc_ref, wr_ref,
              whf_ref, whr_ref, bias_ref, whts_ref, bhts_ref,
              wfc_ref, bfc_ref, out_ref):
    f32 = jnp.float32
    bf16 = jnp.bfloat16
    Bp = out_ref.shape[0]
    H = whf_ref.shape[0]

    rel_proj = (jnp.dot(rel128_ref[...], wr_ref[...],
                        preferred_element_type=f32)
                + bias_ref[...]).astype(bf16)
    wfull = jnp.concatenate([wc_ref[...], rel_proj], axis=0)

    iot = lax.broadcasted_iota(jnp.int32, (Bp, H), 1)
    ohs = []
    for t in range(T):
        r = meta_ref[:, t:t + 1]
        r = jnp.where(r >= 17, r - 17, r)
        ohs.append((r == iot) | (iot == H - 1))
    oh_all = jnp.concatenate(ohs, axis=0).astype(bf16)
    x_all = cpt_ref[...].reshape(T * Bp, H).astype(bf16)
    xcat = jnp.concatenate([x_all, oh_all], axis=1)
    gx_all = jnp.dot(xcat, wfull, preferred_element_type=f32)
    gx = [gx_all[t * Bp:(t + 1) * Bp] for t in range(T)]

    last = jnp.clip(meta_ref[:, T:T + 1], 1, T) - 1
    sel_mask = [(last == t).astype(f32) for t in range(T)]

    def sig2(x):
        return 0.5 * jnp.tanh(x) + 0.5

    def lstm(wh_ref, col0, order):
        h = jnp.zeros((Bp, H), f32)
        c = jnp.zeros((Bp, H), f32)
        sel = jnp.zeros((Bp, H), f32)
        for t in order:
            g = (gx[t][:, col0:col0 + 4 * H]
                 + jnp.dot(h.astype(bf16), wh_ref[...],
                           preferred_element_type=f32))
            i_g = sig2(g[:, 0:H])
            f_g = sig2(g[:, H:2 * H])
            g_g = jnp.tanh(g[:, 2 * H:3 * H])
            o_g = sig2(g[:, 3 * H:4 * H])
            c = f_g * c + i_g * g_g
            h = o_g * jnp.tanh(c)
            sel = sel + sel_mask[t] * h
        return sel

    sel_f = lstm(whf_ref, 0, range(T))
    sel_r = lstm(whr_ref, 4 * H, range(T - 1, -1, -1))

    def leaky(x):
        return jnp.where(x >= 0, x, 0.01 * x)

    hts = leaky(jnp.dot(ht_ref[...].astype(bf16), whts_ref[...],
                        preferred_element_type=f32) + bhts_ref[...])
    cat = jnp.concatenate([hts.astype(bf16), sel_f.astype(bf16),
                           sel_r.astype(bf16)], axis=1)
    fin = jnp.dot(cat, wfc_ref[...], preferred_element_type=f32) + bfc_ref[...]
    out_ref[...] = leaky(fin)


def _encode(cpt_emb, meta, ht_vecs, rel128, wc, wr, whf, whr, bias_mat,
            whts, bhts, wfc, bfc, block_p=1024):
    T, P, D = cpt_emb.shape
    O = wfc.shape[1]
    grid = (P // block_p,)
    full = lambda shape: pl.BlockSpec(shape, lambda i: (0,) * len(shape))
    return pl.pallas_call(
        functools.partial(_enc_body, T),
        grid=grid,
        in_specs=[
            pl.BlockSpec((T, block_p, D), lambda i: (0, i, 0)),
            pl.BlockSpec((block_p, 8), lambda i: (i, 0)),
            pl.BlockSpec((block_p, 2 * D), lambda i: (i, 0)),
            full(rel128.shape), full(wc.shape), full(wr.shape),
            full(whf.shape), full(whr.shape), full(bias_mat.shape),
            full(whts.shape), full(bhts.shape),
            full(wfc.shape), full(bfc.shape),
        ],
        out_specs=pl.BlockSpec((block_p, O), lambda i: (i, 0)),
        out_shape=jax.ShapeDtypeStruct((P, O), jnp.float32),
        compiler_params=pltpu.CompilerParams(
            dimension_semantics=("arbitrary",)),
    )(cpt_emb, meta, ht_vecs, rel128, wc, wr, whf, whr, bias_mat,
      whts, bhts, wfc, bfc)


def kernel(concept_table, relation_table, W_ih_f, W_hh_f, b_ih_f, b_hh_f,
           W_ih_r, W_hh_r, b_ih_r, b_hh_r, W_hts, b_hts, W_fc, b_fc,
           cpt_paths, rel_paths, ht_ids, path_len):
    P, T = cpt_paths.shape
    V, D = concept_table.shape
    H = W_hh_f.shape[1]

    bf16 = jnp.bfloat16
    s4 = jnp.concatenate([jnp.full((H,), 0.5), jnp.full((H,), 0.5),
                          jnp.ones((H,)), jnp.full((H,), 0.5)])[None]
    s8 = jnp.concatenate([s4, s4], axis=1)
    nrel = relation_table.shape[0]
    rel128 = jnp.concatenate(
        [relation_table, jnp.zeros((D - nrel, D), jnp.float32)],
        axis=0).astype(bf16)
    wc = (jnp.concatenate([W_ih_f[:, :D].T, W_ih_r[:, :D].T], axis=1)
          * s8).astype(bf16)
    wr = (jnp.concatenate([W_ih_f[:, D:].T, W_ih_r[:, D:].T], axis=1)
          * s8).astype(bf16)
    whf = (W_hh_f.T * s4).astype(bf16)
    whr = (W_hh_r.T * s4).astype(bf16)
    b8 = jnp.concatenate([(b_ih_f + b_hh_f)[None] * s4,
                          (b_ih_r + b_hh_r)[None] * s4], axis=1)
    bias_mat = jnp.concatenate(
        [jnp.zeros((D - 1, 8 * H), jnp.float32), b8], axis=0)

    K = 4
    Pc = P // K
    outs = []
    for i in range(K):
        sl = slice(i * Pc, (i + 1) * Pc)
        cpt_rows, ht_rows = _sc_gather(
            cpt_paths[sl].astype(jnp.int32).T.reshape(-1),
            ht_ids[sl].astype(jnp.int32).reshape(-1),
            concept_table, Pc, T)
        meta = jnp.concatenate([
            rel_paths[sl].astype(jnp.int32),
            path_len[sl].astype(jnp.int32)[:, None],
            jnp.zeros((Pc, 8 - T - 1), jnp.int32),
        ], axis=1)
        outs.append(_encode(
            cpt_rows.reshape(T, Pc, D), meta, ht_rows.reshape(Pc, 2 * D),
            rel128, wc, wr, whf, whr, bias_mat,
            W_hts.astype(bf16), b_hts[None], W_fc.astype(bf16), b_fc[None]))
    return jnp.concatenate(outs, axis=0)


# speedup vs baseline: 1.3106x; 1.3106x over previous
import functools

import jax
import jax.numpy as jnp
from jax import lax
from jax.experimental import pallas as pl
from jax.experimental.pallas import tpu as pltpu
from jax.experimental.pallas import tpu_sc as plsc

_NC = 2
_NS = 16
_NW = _NC * _NS
_CHUNK = 128


def _sc_gather_body(P, T, cpt_hbm, ht_hbm, table_hbm,
                    out_cpt_hbm, out_ht0_hbm, out_ht1_hbm,
                    idx_v, rows_v, sem, sem2):
    ppw = P // _NW
    npc = ppw // _CHUNK
    ncc = T * npc
    n_chunks = ncc + 2 * npc
    wid = lax.axis_index("s") * _NC + lax.axis_index("c")
    p0 = wid * ppw

    def stage():
        for t in range(T):
            yield (cpt_hbm.at[pl.ds(t * P + p0, ppw)],
                   idx_v.at[pl.ds(t * ppw, ppw)])
        for j in range(2):
            yield (ht_hbm.at[pl.ds(j * P + p0, ppw)],
                   idx_v.at[pl.ds((T + j) * ppw, ppw)])

    for src, dst in stage():
        pltpu.async_copy(src, dst, sem2)
    for src, dst in stage():
        pltpu.make_async_copy(src, dst, sem2).wait()

    def gather(c):
        b = lax.rem(c, 2)
        pltpu.async_copy(table_hbm.at[idx_v.at[pl.ds(c * _CHUNK, _CHUNK)]],
                         rows_v.at[b], sem)

    gather(0)

    def body(c, carry):
        @pl.when(c + 1 < n_chunks)
        def _():
            gather(c + 1)

        b = lax.rem(c, 2)
        pltpu.make_async_copy(
            table_hbm.at[idx_v.at[pl.ds(c * _CHUNK, _CHUNK)]],
            rows_v.at[b], sem).wait()
        pb = lax.rem(c, npc)
        row = p0 + pb * _CHUNK

        @pl.when(c < ncc)
        def _():
            pltpu.sync_copy(rows_v.at[b],
                            out_cpt_hbm.at[c // npc, pl.ds(row, _CHUNK)])

        @pl.when((c >= ncc) & (c < ncc + npc))
        def _():
            pltpu.sync_copy(rows_v.at[b], out_ht0_hbm.at[pl.ds(row, _CHUNK)])

        @pl.when(c >= ncc + npc)
        def _():
            pltpu.sync_copy(rows_v.at[b], out_ht1_hbm.at[pl.ds(row, _CHUNK)])

        return carry

    lax.fori_loop(0, n_chunks, body, 0)


def _sc_gather(cpt_tm, ht_tm, table, P, T):
    D = table.shape[1]
    ppw = P // _NW
    assert ppw % _CHUNK == 0 and (T * ppw) % _CHUNK == 0
    mesh = plsc.VectorSubcoreMesh(core_axis_name="c", subcore_axis_name="s")
    k = functools.partial(
        pl.kernel,
        mesh=mesh,
        out_type=(
            jax.ShapeDtypeStruct((T, P, D), table.dtype),
            jax.ShapeDtypeStruct((P, D), table.dtype),
            jax.ShapeDtypeStruct((P, D), table.dtype),
        ),
        scratch_types=[
            pltpu.VMEM(((T + 2) * ppw,), jnp.int32),
            pltpu.VMEM((2, _CHUNK, D), table.dtype),
            pltpu.SemaphoreType.DMA,
            pltpu.SemaphoreType.DMA,
        ],
    )(functools.partial(_sc_gather_body, P, T))
    return k(cpt_tm, ht_tm, table)


def _enc_body(T, cpt_ref, meta_ref, ht0_ref, ht1_ref, rel128_ref, wc_ref, wr_ref,
              whf_ref, whr_ref, bias_ref, whts_ref, bhts_ref,
              wfc_ref, bfc_ref, out_ref):
    f32 = jnp.float32
    bf16 = jnp.bfloat16
    Bp = out_ref.shape[0]
    H = whf_ref.shape[0]

    rel_proj = (jnp.dot(rel128_ref[...], wr_ref[...],
                        preferred_element_type=f32)
                + bias_ref[...]).astype(bf16)
    wfull = jnp.concatenate([wc_ref[...], rel_proj], axis=0)

    iot = lax.broadcasted_iota(jnp.int32, (Bp, H), 1)
    ohs = []
    for t in range(T):
        r = meta_ref[:, t:t + 1]
        r = jnp.where(r >= 17, r - 17, r)
        ohs.append((r == iot) | (iot == H - 1))
    oh_all = jnp.concatenate(ohs, axis=0).astype(bf16)
    x_all = cpt_ref[...].reshape(T * Bp, H).astype(bf16)
    xcat = jnp.concatenate([x_all, oh_all], axis=1)
    gx_all = jnp.dot(xcat, wfull, preferred_element_type=f32)
    gx = [gx_all[t * Bp:(t + 1) * Bp] for t in range(T)]

    last = jnp.clip(meta_ref[:, T:T + 1], 1, T) - 1
    sel_mask = [(last == t).astype(f32) for t in range(T)]

    def sig2(x):
        return 0.5 * jnp.tanh(x) + 0.5

    def lstm(wh_ref, col0, order):
        h = jnp.zeros((Bp, H), f32)
        c = jnp.zeros((Bp, H), f32)
        sel = jnp.zeros((Bp, H), f32)
        for t in order:
            g = (gx[t][:, col0:col0 + 4 * H]
                 + jnp.dot(h.astype(bf16), wh_ref[...],
                           preferred_element_type=f32))
            i_g = sig2(g[:, 0:H])
            f_g = sig2(g[:, H:2 * H])
            g_g = jnp.tanh(g[:, 2 * H:3 * H])
            o_g = sig2(g[:, 3 * H:4 * H])
            c = f_g * c + i_g * g_g
            h = o_g * jnp.tanh(c)
            sel = sel + sel_mask[t] * h
        return sel

    sel_f = lstm(whf_ref, 0, range(T))
    sel_r = lstm(whr_ref, 4 * H, range(T - 1, -1, -1))

    def leaky(x):
        return jnp.where(x >= 0, x, 0.01 * x)

    hcat = jnp.concatenate([ht0_ref[...].astype(bf16),
                            ht1_ref[...].astype(bf16)], axis=1)
    hts = leaky(jnp.dot(hcat, whts_ref[...],
                        preferred_element_type=f32) + bhts_ref[...])
    cat = jnp.concatenate([hts.astype(bf16), sel_f.astype(bf16),
                           sel_r.astype(bf16)], axis=1)
    fin = jnp.dot(cat, wfc_ref[...], preferred_element_type=f32) + bfc_ref[...]
    out_ref[...] = leaky(fin)


def _encode(cpt_emb, meta, ht0, ht1, rel128, wc, wr, whf, whr, bias_mat,
            whts, bhts, wfc, bfc, block_p=1024):
    T, P, D = cpt_emb.shape
    O = wfc.shape[1]
    grid = (P // block_p,)
    full = lambda shape: pl.BlockSpec(shape, lambda i: (0,) * len(shape))
    return pl.pallas_call(
        functools.partial(_enc_body, T),
        grid=grid,
        in_specs=[
            pl.BlockSpec((T, block_p, D), lambda i: (0, i, 0)),
            pl.BlockSpec((block_p, 8), lambda i: (i, 0)),
            pl.BlockSpec((block_p, D), lambda i: (i, 0)),
            pl.BlockSpec((block_p, D), lambda i: (i, 0)),
            full(rel128.shape), full(wc.shape), full(wr.shape),
            full(whf.shape), full(whr.shape), full(bias_mat.shape),
            full(whts.shape), full(bhts.shape),
            full(wfc.shape), full(bfc.shape),
        ],
        out_specs=pl.BlockSpec((block_p, O), lambda i: (i, 0)),
        out_shape=jax.ShapeDtypeStruct((P, O), jnp.float32),
        compiler_params=pltpu.CompilerParams(
            dimension_semantics=("arbitrary",)),
    )(cpt_emb, meta, ht0, ht1, rel128, wc, wr, whf, whr, bias_mat,
      whts, bhts, wfc, bfc)


def kernel(concept_table, relation_table, W_ih_f, W_hh_f, b_ih_f, b_hh_f,
           W_ih_r, W_hh_r, b_ih_r, b_hh_r, W_hts, b_hts, W_fc, b_fc,
           cpt_paths, rel_paths, ht_ids, path_len):
    P, T = cpt_paths.shape
    V, D = concept_table.shape
    H = W_hh_f.shape[1]

    bf16 = jnp.bfloat16
    s4 = jnp.concatenate([jnp.full((H,), 0.5), jnp.full((H,), 0.5),
                          jnp.ones((H,)), jnp.full((H,), 0.5)])[None]
    s8 = jnp.concatenate([s4, s4], axis=1)
    nrel = relation_table.shape[0]
    rel128 = jnp.concatenate(
        [relation_table, jnp.zeros((D - nrel, D), jnp.float32)],
        axis=0).astype(bf16)
    wc = (jnp.concatenate([W_ih_f[:, :D].T, W_ih_r[:, :D].T], axis=1)
          * s8).astype(bf16)
    wr = (jnp.concatenate([W_ih_f[:, D:].T, W_ih_r[:, D:].T], axis=1)
          * s8).astype(bf16)
    whf = (W_hh_f.T * s4).astype(bf16)
    whr = (W_hh_r.T * s4).astype(bf16)
    b8 = jnp.concatenate([(b_ih_f + b_hh_f)[None] * s4,
                          (b_ih_r + b_hh_r)[None] * s4], axis=1)
    bias_mat = jnp.concatenate(
        [jnp.zeros((D - 1, 8 * H), jnp.float32), b8], axis=0)

    K = 4
    Pc = P // K
    outs = []
    for i in range(K):
        sl = slice(i * Pc, (i + 1) * Pc)
        cpt_emb, ht0, ht1 = _sc_gather(
            cpt_paths[sl].astype(jnp.int32).T.reshape(-1),
            ht_ids[sl].astype(jnp.int32).T.reshape(-1),
            concept_table, Pc, T)
        meta = jnp.concatenate([
            rel_paths[sl].astype(jnp.int32),
            path_len[sl].astype(jnp.int32)[:, None],
            jnp.zeros((Pc, 8 - T - 1), jnp.int32),
        ], axis=1)
        outs.append(_encode(
            cpt_emb, meta, ht0, ht1,
            rel128, wc, wr, whf, whr, bias_mat,
            W_hts.astype(bf16), b_hts[None], W_fc.astype(bf16), b_fc[None]))
    return jnp.concatenate(outs, axis=0)
